# Initial kernel scaffold; baseline (speedup 1.0000x reference)
import jax
import jax.numpy as jnp
from jax.experimental import pallas as pl


def kernel(node_features, neigh_idx, W1, b1, W2, b2):
    raise NotImplementedError("write your pallas kernel here")


# speedup vs baseline: 1.3970x; 1.3970x over previous
import functools

import jax
import jax.numpy as jnp
from jax import lax
from jax.experimental import pallas as pl
from jax.experimental.pallas import tpu as pltpu
from jax.experimental.pallas import tpu_sc as plsc

N = 10000
D = 128
K = 32

NW = 32
NP = 10240
C = NP // NW
NB = 8
NCHUNK = C // NB
VPR = D // 16


def _sc_gather_mean_body(x_hbm, idx_hbm, out_hbm, idx_v, rows0, rows1,
                         outb0, outb1, sg0, sg1, so0, so1):
    wid = lax.axis_index("s") * 2 + lax.axis_index("c")
    node_base = wid * C
    pltpu.sync_copy(idx_hbm.at[pl.ds(node_base * K, C * K)], idx_v)

    rows = (rows0, rows1)
    outb = (outb0, outb1)
    sg = (sg0, sg1)
    so = (so0, so1)

    def gather(gi, b):
        return pltpu.make_async_copy(
            x_hbm.at[idx_v.at[pl.ds(gi * (NB * K), NB * K)]], rows[b], sg[b])

    def outcopy(gi, b):
        return pltpu.make_async_copy(
            outb[b], out_hbm.at[pl.ds(node_base + gi * NB, NB)], so[b])

    gather(0, 0).start()
    gather(1, 1).start()

    @pl.loop(0, NCHUNK, step=2)
    def _chunks(g):
        for b in range(2):
            gi = g + b
            gather(gi, b).wait()
            @pl.loop(0, NB)
            def _node(j):
                accs = [jnp.zeros((16,), jnp.float32) for _ in range(VPR)]
                for k in range(K):
                    for dd in range(VPR):
                        accs[dd] = accs[dd] + rows[b][j * K + k,
                                                      pl.ds(dd * 16, 16)]
                for dd in range(VPR):
                    outb[b][j, pl.ds(dd * 16, 16)] = accs[dd] * (1.0 / K)
            @pl.when(gi + 2 < NCHUNK)
            def _():
                gather(gi + 2, b).start()
            outcopy(gi, b).start()
            outcopy(gi, b).wait()


@functools.partial(jax.jit, static_argnames=())
def _sc_gather_mean(x, idx_flat):
    kern = pl.kernel(
        _sc_gather_mean_body,
        out_type=jax.ShapeDtypeStruct((NP, D), jnp.float32),
        mesh=plsc.VectorSubcoreMesh(core_axis_name="c", subcore_axis_name="s"),
        scratch_types=[
            pltpu.VMEM((C * K,), jnp.int32),
            pltpu.VMEM((NB * K, D), jnp.float32),
            pltpu.VMEM((NB * K, D), jnp.float32),
            pltpu.VMEM((NB, D), jnp.float32),
            pltpu.VMEM((NB, D), jnp.float32),
            pltpu.SemaphoreType.DMA,
            pltpu.SemaphoreType.DMA,
            pltpu.SemaphoreType.DMA,
            pltpu.SemaphoreType.DMA,
        ],
    )
    return kern(x, idx_flat)


def _linear_body(x_ref, a_ref, w_ref, b_ref, o_ref, *, act):
    wa = w_ref[:, :D]
    wb = w_ref[:, D:]
    acc = lax.dot_general(x_ref[...], wa, (((1,), (1,)), ((), ())),
                          preferred_element_type=jnp.float32)
    acc = acc + lax.dot_general(a_ref[...], wb, (((1,), (1,)), ((), ())),
                                preferred_element_type=jnp.float32)
    acc = acc + b_ref[...]
    if act:
        acc = jnp.maximum(acc, 0.0)
    o_ref[...] = acc


def _linear(x, agg, w, b, act):
    BM = 1000
    grid = (N // BM,)
    return pl.pallas_call(
        functools.partial(_linear_body, act=act),
        out_shape=jax.ShapeDtypeStruct((N, D), jnp.float32),
        grid=grid,
        in_specs=[
            pl.BlockSpec((BM, D), lambda i: (i, 0)),
            pl.BlockSpec((BM, D), lambda i: (i, 0)),
            pl.BlockSpec((D, 2 * D), lambda i: (0, 0)),
            pl.BlockSpec((1, D), lambda i: (0, 0)),
        ],
        out_specs=pl.BlockSpec((BM, D), lambda i: (i, 0)),
    )(x, agg, w, b)


def kernel(node_features, neigh_idx, W1, b1, W2, b2):
    idx = neigh_idx.astype(jnp.int32).reshape(N * K)
    idx = jnp.concatenate([idx, jnp.zeros((NP * K - N * K,), jnp.int32)])
    b1r = b1.reshape(1, D)
    b2r = b2.reshape(1, D)

    agg1 = _sc_gather_mean(node_features, idx)[:N]
    h = _linear(node_features, agg1, W1, b1r, act=True)
    agg2 = _sc_gather_mean(h, idx)[:N]
    out = _linear(h, agg2, W2, b2r, act=False)
    return out


# speedup vs baseline: 1.3991x; 1.0015x over previous
import functools

import jax
import jax.numpy as jnp
from jax import lax
from jax.experimental import pallas as pl
from jax.experimental.pallas import tpu as pltpu
from jax.experimental.pallas import tpu_sc as plsc

N = 10000
D = 128
K = 32

NW = 32
NP = 10240
C = NP // NW
NB = 4
NBUF = 4
NCHUNK = C // NB
VPR = D // 16


def _sc_gather_mean_body(x_hbm, idx_hbm, out_hbm, idx_v, rows0, rows1,
                         rows2, rows3, outb, sg0, sg1, sg2, sg3, so):
    wid = lax.axis_index("s") * 2 + lax.axis_index("c")
    node_base = wid * C
    pltpu.sync_copy(idx_hbm.at[pl.ds(node_base * K, C * K)], idx_v)

    rows = (rows0, rows1, rows2, rows3)
    sg = (sg0, sg1, sg2, sg3)

    def gather(gi, b):
        return pltpu.make_async_copy(
            x_hbm.at[idx_v.at[pl.ds(gi * (NB * K), NB * K)]], rows[b], sg[b])

    for b in range(NBUF):
        gather(b, b).start()

    @pl.loop(0, NCHUNK, step=NBUF)
    def _chunks(g):
        for b in range(NBUF):
            gi = g + b
            gather(gi, b).wait()
            @pl.loop(0, NB)
            def _node(j):
                accs = [jnp.zeros((16,), jnp.float32) for _ in range(VPR)]
                for k in range(K):
                    for dd in range(VPR):
                        accs[dd] = accs[dd] + rows[b][j * K + k,
                                                      pl.ds(dd * 16, 16)]
                for dd in range(VPR):
                    outb[gi * NB + j, pl.ds(dd * 16, 16)] = \
                        accs[dd] * (1.0 / K)
            @pl.when(gi + NBUF < NCHUNK)
            def _():
                gather(gi + NBUF, b).start()

    pltpu.make_async_copy(outb, out_hbm.at[pl.ds(node_base, C)], so).start()
    pltpu.make_async_copy(outb, out_hbm.at[pl.ds(node_base, C)], so).wait()


@functools.partial(jax.jit, static_argnames=())
def _sc_gather_mean(x, idx_flat):
    kern = pl.kernel(
        _sc_gather_mean_body,
        out_type=jax.ShapeDtypeStruct((NP, D), jnp.float32),
        mesh=plsc.VectorSubcoreMesh(core_axis_name="c", subcore_axis_name="s"),
        scratch_types=[
            pltpu.VMEM((C * K,), jnp.int32),
            pltpu.VMEM((NB * K, D), jnp.float32),
            pltpu.VMEM((NB * K, D), jnp.float32),
            pltpu.VMEM((NB * K, D), jnp.float32),
            pltpu.VMEM((NB * K, D), jnp.float32),
            pltpu.VMEM((C, D), jnp.float32),
            pltpu.SemaphoreType.DMA,
            pltpu.SemaphoreType.DMA,
            pltpu.SemaphoreType.DMA,
            pltpu.SemaphoreType.DMA,
            pltpu.SemaphoreType.DMA,
        ],
    )
    return kern(x, idx_flat)


def _linear_body(x_ref, a_ref, w_ref, b_ref, o_ref, *, act):
    wa = w_ref[:, :D]
    wb = w_ref[:, D:]
    acc = lax.dot_general(x_ref[...], wa, (((1,), (1,)), ((), ())),
                          preferred_element_type=jnp.float32)
    acc = acc + lax.dot_general(a_ref[...], wb, (((1,), (1,)), ((), ())),
                                preferred_element_type=jnp.float32)
    acc = acc + b_ref[...]
    if act:
        acc = jnp.maximum(acc, 0.0)
    o_ref[...] = acc


def _linear(x, agg, w, b, act):
    BM = 1000
    grid = (N // BM,)
    return pl.pallas_call(
        functools.partial(_linear_body, act=act),
        out_shape=jax.ShapeDtypeStruct((N, D), jnp.float32),
        grid=grid,
        in_specs=[
            pl.BlockSpec((BM, D), lambda i: (i, 0)),
            pl.BlockSpec((BM, D), lambda i: (i, 0)),
            pl.BlockSpec((D, 2 * D), lambda i: (0, 0)),
            pl.BlockSpec((1, D), lambda i: (0, 0)),
        ],
        out_specs=pl.BlockSpec((BM, D), lambda i: (i, 0)),
    )(x, agg, w, b)


def kernel(node_features, neigh_idx, W1, b1, W2, b2):
    idx = neigh_idx.astype(jnp.int32).reshape(N * K)
    idx = jnp.concatenate([idx, jnp.zeros((NP * K - N * K,), jnp.int32)])
    b1r = b1.reshape(1, D)
    b2r = b2.reshape(1, D)

    agg1 = _sc_gather_mean(node_features, idx)[:N]
    h = _linear(node_features, agg1, W1, b1r, act=True)
    agg2 = _sc_gather_mean(h, idx)[:N]
    out = _linear(h, agg2, W2, b2r, act=False)
    return out


# speedup vs baseline: 6.2311x; 4.4536x over previous
import functools

import jax
import jax.numpy as jnp
from jax import lax
from jax.experimental import pallas as pl
from jax.experimental.pallas import tpu as pltpu
from jax.experimental.pallas import tpu_sc as plsc

N = 10000
D = 128
K = 32

NW = 32
NP = 10240
C = NP // NW
NB = 4
NBUF = 2
NCHUNK = C // NB
VPR = D // 16


def _sc_gather_mean_body(x_hbm, idx_hbm, out_hbm, x_spmem, idx_v, rows0,
                         rows1, outb0, outb1, sg0, sg1, so0, so1):
    wid = lax.axis_index("s") * 2 + lax.axis_index("c")
    node_base = wid * C
    pltpu.sync_copy(idx_hbm.at[pl.ds(node_base * K, C * K)], idx_v)

    sid = lax.axis_index("s")
    rpt = 624
    pltpu.sync_copy(x_hbm.at[pl.ds(sid * rpt, rpt)],
                    x_spmem.at[pl.ds(sid * rpt, rpt)])
    @pl.when(sid == 0)
    def _():
        pltpu.sync_copy(x_hbm.at[pl.ds(16 * rpt, N - 16 * rpt)],
                        x_spmem.at[pl.ds(16 * rpt, N - 16 * rpt)])
    plsc.subcore_barrier()

    rows = (rows0, rows1)
    outb = (outb0, outb1)
    sg = (sg0, sg1)
    so = (so0, so1)

    def gather(gi, b):
        return pltpu.make_async_copy(
            x_spmem.at[idx_v.at[pl.ds(gi * (NB * K), NB * K)]], rows[b],
            sg[b])

    def outcopy(gi, b):
        return pltpu.make_async_copy(
            outb[b], out_hbm.at[pl.ds(node_base + gi * NB, NB)], so[b])

    for b in range(NBUF):
        gather(b, b).start()

    @pl.loop(0, NCHUNK, step=NBUF)
    def _chunks(g):
        for b in range(NBUF):
            gi = g + b
            gather(gi, b).wait()
            @pl.when(gi >= NBUF)
            def _():
                outcopy(gi, b).wait()
            @pl.loop(0, NB)
            def _node(j):
                accs = [jnp.zeros((16,), jnp.float32) for _ in range(VPR)]
                for k in range(K):
                    for dd in range(VPR):
                        accs[dd] = accs[dd] + rows[b][j * K + k,
                                                      pl.ds(dd * 16, 16)]
                for dd in range(VPR):
                    outb[b][j, pl.ds(dd * 16, 16)] = accs[dd] * (1.0 / K)
            @pl.when(gi + NBUF < NCHUNK)
            def _():
                gather(gi + NBUF, b).start()
            outcopy(gi, b).start()

    for b in range(NBUF):
        outcopy(NCHUNK - NBUF + b, b).wait()


@functools.partial(jax.jit, static_argnames=())
def _sc_gather_mean(x, idx_flat):
    kern = pl.kernel(
        _sc_gather_mean_body,
        out_type=jax.ShapeDtypeStruct((NP, D), jnp.float32),
        mesh=plsc.VectorSubcoreMesh(core_axis_name="c", subcore_axis_name="s"),
        scratch_types=[
            pltpu.VMEM_SHARED((N, D), jnp.float32),
            pltpu.VMEM((C * K,), jnp.int32),
            pltpu.VMEM((NB * K, D), jnp.float32),
            pltpu.VMEM((NB * K, D), jnp.float32),
            pltpu.VMEM((NB, D), jnp.float32),
            pltpu.VMEM((NB, D), jnp.float32),
            pltpu.SemaphoreType.DMA,
            pltpu.SemaphoreType.DMA,
            pltpu.SemaphoreType.DMA,
            pltpu.SemaphoreType.DMA,
        ],
    )
    return kern(x, idx_flat)


def _linear_body(x_ref, a_ref, w_ref, b_ref, o_ref, *, act):
    wa = w_ref[:, :D]
    wb = w_ref[:, D:]
    acc = lax.dot_general(x_ref[...], wa, (((1,), (1,)), ((), ())),
                          preferred_element_type=jnp.float32)
    acc = acc + lax.dot_general(a_ref[...], wb, (((1,), (1,)), ((), ())),
                                preferred_element_type=jnp.float32)
    acc = acc + b_ref[...]
    if act:
        acc = jnp.maximum(acc, 0.0)
    o_ref[...] = acc


def _linear(x, agg, w, b, act):
    BM = 1000
    grid = (N // BM,)
    return pl.pallas_call(
        functools.partial(_linear_body, act=act),
        out_shape=jax.ShapeDtypeStruct((N, D), jnp.float32),
        grid=grid,
        in_specs=[
            pl.BlockSpec((BM, D), lambda i: (i, 0)),
            pl.BlockSpec((BM, D), lambda i: (i, 0)),
            pl.BlockSpec((D, 2 * D), lambda i: (0, 0)),
            pl.BlockSpec((1, D), lambda i: (0, 0)),
        ],
        out_specs=pl.BlockSpec((BM, D), lambda i: (i, 0)),
    )(x, agg, w, b)


def kernel(node_features, neigh_idx, W1, b1, W2, b2):
    idx = neigh_idx.astype(jnp.int32).reshape(N * K)
    idx = jnp.concatenate([idx, jnp.zeros((NP * K - N * K,), jnp.int32)])
    b1r = b1.reshape(1, D)
    b2r = b2.reshape(1, D)

    agg1 = _sc_gather_mean(node_features, idx)[:N]
    h = _linear(node_features, agg1, W1, b1r, act=True)
    agg2 = _sc_gather_mean(h, idx)[:N]
    out = _linear(h, agg2, W2, b2r, act=False)
    return out


# speedup vs baseline: 6.5107x; 1.0449x over previous
import functools

import jax
import jax.numpy as jnp
from jax import lax
from jax.experimental import pallas as pl
from jax.experimental.pallas import tpu as pltpu
from jax.experimental.pallas import tpu_sc as plsc

N = 10000
D = 128
K = 32

NW = 32
NP = 10240
C = NP // NW
NB = 4
NBUF = 2
NCHUNK = C // NB
VPR = D // 16


def _sc_gather_mean_body(x_hbm, idx_hbm, out_hbm, x_spmem, idx_v, rows0,
                         rows1, outb0, outb1, sg0, sg1, so0, so1):
    wid = lax.axis_index("s") * 2 + lax.axis_index("c")
    node_base = wid * C
    nchunk_w = jnp.minimum(NCHUNK, (N - node_base) // NB)
    stage_base = jnp.minimum(node_base * K, N * K - C * K)
    idx_off = node_base * K - stage_base
    pltpu.sync_copy(idx_hbm.at[pl.ds(stage_base, C * K)], idx_v)

    sid = lax.axis_index("s")
    rpt = 624
    pltpu.sync_copy(x_hbm.at[pl.ds(sid * rpt, rpt)],
                    x_spmem.at[pl.ds(sid * rpt, rpt)])
    @pl.when(sid == 0)
    def _():
        pltpu.sync_copy(x_hbm.at[pl.ds(16 * rpt, N - 16 * rpt)],
                        x_spmem.at[pl.ds(16 * rpt, N - 16 * rpt)])
    plsc.subcore_barrier()

    rows = (rows0, rows1)
    outb = (outb0, outb1)
    sg = (sg0, sg1)
    so = (so0, so1)

    def gather(gi, b):
        off = jnp.minimum(idx_off + gi * (NB * K), C * K - NB * K)
        return pltpu.make_async_copy(
            x_spmem.at[idx_v.at[pl.ds(off, NB * K)]], rows[b], sg[b])

    def outcopy(gi, b):
        row = jnp.minimum(node_base + gi * NB, N - NB)
        return pltpu.make_async_copy(
            outb[b], out_hbm.at[pl.ds(row, NB)], so[b])

    for b in range(NBUF):
        gather(b, b).start()

    @pl.loop(0, NCHUNK, step=NBUF)
    def _chunks(g):
        for b in range(NBUF):
            gi = g + b
            gather(gi, b).wait()
            @pl.when((gi >= NBUF) & (gi < nchunk_w))
            def _():
                outcopy(gi, b).wait()
            @pl.loop(0, NB)
            def _node(j):
                accs = [jnp.zeros((16,), jnp.float32) for _ in range(VPR)]
                for k in range(K):
                    for dd in range(VPR):
                        accs[dd] = accs[dd] + rows[b][j * K + k,
                                                      pl.ds(dd * 16, 16)]
                for dd in range(VPR):
                    outb[b][j, pl.ds(dd * 16, 16)] = accs[dd] * (1.0 / K)
            @pl.when(gi + NBUF < NCHUNK)
            def _():
                gather(gi + NBUF, b).start()
            @pl.when(gi < nchunk_w)
            def _():
                outcopy(gi, b).start()

    for b in range(NBUF):
        outcopy(nchunk_w - NBUF + b, b).wait()


@functools.partial(jax.jit, static_argnames=())
def _sc_gather_mean(x, idx_flat):
    kern = pl.kernel(
        _sc_gather_mean_body,
        out_type=jax.ShapeDtypeStruct((N, D), jnp.float32),
        mesh=plsc.VectorSubcoreMesh(core_axis_name="c", subcore_axis_name="s"),
        scratch_types=[
            pltpu.VMEM_SHARED((N, D), jnp.float32),
            pltpu.VMEM((C * K,), jnp.int32),
            pltpu.VMEM((NB * K, D), jnp.float32),
            pltpu.VMEM((NB * K, D), jnp.float32),
            pltpu.VMEM((NB, D), jnp.float32),
            pltpu.VMEM((NB, D), jnp.float32),
            pltpu.SemaphoreType.DMA,
            pltpu.SemaphoreType.DMA,
            pltpu.SemaphoreType.DMA,
            pltpu.SemaphoreType.DMA,
        ],
    )
    return kern(x, idx_flat)


def _linear_body(x_ref, a_ref, w_ref, b_ref, o_ref, *, act):
    wa = w_ref[:, :D]
    wb = w_ref[:, D:]
    acc = lax.dot_general(x_ref[...], wa, (((1,), (1,)), ((), ())),
                          preferred_element_type=jnp.float32)
    acc = acc + lax.dot_general(a_ref[...], wb, (((1,), (1,)), ((), ())),
                                preferred_element_type=jnp.float32)
    acc = acc + b_ref[...]
    if act:
        acc = jnp.maximum(acc, 0.0)
    o_ref[...] = acc


def _linear(x, agg, w, b, act):
    BM = 1000
    grid = (N // BM,)
    return pl.pallas_call(
        functools.partial(_linear_body, act=act),
        out_shape=jax.ShapeDtypeStruct((N, D), jnp.float32),
        grid=grid,
        in_specs=[
            pl.BlockSpec((BM, D), lambda i: (i, 0)),
            pl.BlockSpec((BM, D), lambda i: (i, 0)),
            pl.BlockSpec((D, 2 * D), lambda i: (0, 0)),
            pl.BlockSpec((1, D), lambda i: (0, 0)),
        ],
        out_specs=pl.BlockSpec((BM, D), lambda i: (i, 0)),
    )(x, agg, w, b)


def kernel(node_features, neigh_idx, W1, b1, W2, b2):
    idx = neigh_idx.astype(jnp.int32).reshape(N * K)
    b1r = b1.reshape(1, D)
    b2r = b2.reshape(1, D)

    agg1 = _sc_gather_mean(node_features, idx)
    h = _linear(node_features, agg1, W1, b1r, act=True)
    agg2 = _sc_gather_mean(h, idx)
    out = _linear(h, agg2, W2, b2r, act=False)
    return out
